# baseline (speedup 1.0000x reference)
import functools

import jax
import jax.numpy as jnp
from jax import lax
from jax.experimental import pallas as pl
from jax.experimental.pallas import tpu as pltpu
from jax.experimental.pallas import tpu_sc as plsc

BATCH = 16384
FACTORS = 64
MLP_FACTOR = 32
NC, NS = 2, 16
NW = NC * NS
BPW = BATCH // NW

NPASS = 4
PB = BPW // NPASS
CHUNK = 16
NCHP = PB // CHUNK
DEPTH = 2
VL = 16


def _sc_gather(user_ids, item_ids, gu_tab, gi_tab, mu_tab, mi_tab):
    mesh = plsc.VectorSubcoreMesh(core_axis_name="c", subcore_axis_name="s")
    out_type = (
        jax.ShapeDtypeStruct((BATCH // 2, 128), jnp.float32),
        jax.ShapeDtypeStruct((BATCH // 2, 128), jnp.float32),
        jax.ShapeDtypeStruct((BATCH // 4, 128), jnp.float32),
        jax.ShapeDtypeStruct((BATCH // 4, 128), jnp.float32),
    )

    @functools.partial(
        pl.kernel,
        mesh=mesh,
        out_type=out_type,
        scratch_types=[
            pltpu.VMEM((BPW,), jnp.int32),
            pltpu.VMEM((BPW,), jnp.int32),
            pltpu.VMEM((PB, FACTORS), jnp.float32),
            pltpu.VMEM((PB, FACTORS), jnp.float32),
            pltpu.VMEM((PB, MLP_FACTOR), jnp.float32),
            pltpu.VMEM((PB, MLP_FACTOR), jnp.float32),
            pltpu.VMEM((PB // 2, 128), jnp.float32),
            pltpu.VMEM((PB // 2, 128), jnp.float32),
            pltpu.VMEM((PB // 4, 128), jnp.float32),
            pltpu.VMEM((PB // 4, 128), jnp.float32),
            pltpu.SemaphoreType.DMA,
            pltpu.SemaphoreType.DMA,
        ],
    )
    def k(uid_hbm, iid_hbm, gu_hbm, gi_hbm, mu_hbm, mi_hbm,
          ogu_hbm, ogi_hbm, omu_hbm, omi_hbm,
          us_v, is_v, agu, agi, amu, ami, bgu, bgi, bmu, bmi, gsem, osem):
        wid = lax.axis_index("s") * NC + lax.axis_index("c")
        base = wid * BPW
        pltpu.sync_copy(uid_hbm.at[pl.ds(base, BPW)], us_v)
        pltpu.sync_copy(iid_hbm.at[pl.ds(base, BPW)], is_v)

        def drain_chunk():
            for _ in range(CHUNK):
                pltpu.make_async_copy(gu_hbm.at[pl.ds(0, 1), :],
                                      agu.at[pl.ds(0, 1), :], gsem).wait()
                pltpu.make_async_copy(mu_hbm.at[pl.ds(0, 1), :],
                                      amu.at[pl.ds(0, 1), :], gsem).wait()
                pltpu.make_async_copy(gi_hbm.at[pl.ds(0, 1), :],
                                      agi.at[pl.ds(0, 1), :], gsem).wait()
                pltpu.make_async_copy(mi_hbm.at[pl.ds(0, 1), :],
                                      ami.at[pl.ds(0, 1), :], gsem).wait()

        for p in range(NPASS):
            poff = p * PB

            @pl.loop(0, NCHP)
            def _(c):
                roff = poff + c * CHUNK
                uvec = us_v[pl.ds(roff, CHUNK)]
                ivec = is_v[pl.ds(roff, CHUNK)]
                for j in range(CHUNK):
                    i = c * CHUNK + j
                    u = uvec[j]
                    it = ivec[j]
                    pltpu.async_copy(gu_hbm.at[pl.ds(u, 1), :],
                                     agu.at[pl.ds(i, 1), :], gsem)
                    pltpu.async_copy(mu_hbm.at[pl.ds(u, 1), :],
                                     amu.at[pl.ds(i, 1), :], gsem)
                    pltpu.async_copy(gi_hbm.at[pl.ds(it, 1), :],
                                     agi.at[pl.ds(i, 1), :], gsem)
                    pltpu.async_copy(mi_hbm.at[pl.ds(it, 1), :],
                                     ami.at[pl.ds(i, 1), :], gsem)

                @pl.when(c >= DEPTH)
                def _():
                    drain_chunk()

            for _ in range(min(DEPTH, NCHP)):
                drain_chunk()

            @pl.loop(0, PB // 2)
            def _(q):
                for a, b in ((agu, bgu), (agi, bgi)):
                    for h in range(2):
                        for j in range(FACTORS // VL):
                            b[q, pl.ds(64 * h + VL * j, VL)] = (
                                a[2 * q + h, pl.ds(VL * j, VL)])

            @pl.loop(0, PB // 4)
            def _(q):
                for a, b in ((amu, bmu), (ami, bmi)):
                    for kk in range(4):
                        for j in range(MLP_FACTOR // VL):
                            b[q, pl.ds(32 * kk + VL * j, VL)] = (
                                a[4 * q + kk, pl.ds(VL * j, VL)])

            gr = pl.multiple_of((base + poff) // 2, PB // 2)
            mr = pl.multiple_of((base + poff) // 4, PB // 4)
            o0 = pltpu.async_copy(bgu, ogu_hbm.at[pl.ds(gr, PB // 2), :], osem)
            o1 = pltpu.async_copy(bgi, ogi_hbm.at[pl.ds(gr, PB // 2), :], osem)
            o2 = pltpu.async_copy(bmu, omu_hbm.at[pl.ds(mr, PB // 4), :], osem)
            o3 = pltpu.async_copy(bmi, omi_hbm.at[pl.ds(mr, PB // 4), :], osem)
            o0.wait()
            o1.wait()
            o2.wait()
            o3.wait()

    return k(user_ids, item_ids, gu_tab, gi_tab, mu_tab, mi_tab)


SBLK = 2048
GROWS = SBLK // 2
MROWS = SBLK // 4


def _dense_body(gu_ref, gi_ref, mu_ref, mi_ref, w0u, w0i, b0r, w1, b1r,
                w2, b2r, whb, wg2, og_ref, om_ref):
    gprod = gu_ref[...] * gi_ref[...]
    og_ref[...] = gprod @ wg2[...]
    h = jnp.maximum(mu_ref[...] @ w0u[...] + mi_ref[...] @ w0i[...] + b0r[...],
                    0.0)
    h = jnp.maximum(h @ w1[...] + b1r[...], 0.0)
    h = jnp.maximum(h @ w2[...] + b2r[...], 0.0)
    om_ref[...] = h @ whb[...]


def _block_diag4(w):
    r, c = w.shape
    z = jnp.zeros((r, c), w.dtype)
    return jnp.block([[w if i == j else z for j in range(4)]
                      for i in range(4)])


def _tc_dense(gu2, gi2, mu2, mi2, W0, b0, W1, b1, W2, b2, Wout, bout):
    grid = (BATCH // SBLK,)
    w0u = _block_diag4(W0[:MLP_FACTOR])
    w0i = _block_diag4(W0[MLP_FACTOR:])
    w1b = _block_diag4(W1)
    w2b = _block_diag4(W2)
    whb = _block_diag4(Wout[FACTORS:])
    wg = Wout[:FACTORS]
    z = jnp.zeros((FACTORS, 1), wg.dtype)
    wg2 = jnp.block([[wg, z], [z, wg]])
    b0r = jnp.tile(b0, 4).reshape(1, -1)
    b1r = jnp.tile(b1, 4).reshape(1, -1)
    b2r = jnp.tile(b2, 4).reshape(1, -1)

    def full(a):
        return pl.BlockSpec(a.shape, lambda i: (0,) * a.ndim)

    og, om = pl.pallas_call(
        _dense_body,
        grid=grid,
        in_specs=[
            pl.BlockSpec((GROWS, 128), lambda i: (i, 0)),
            pl.BlockSpec((GROWS, 128), lambda i: (i, 0)),
            pl.BlockSpec((MROWS, 128), lambda i: (i, 0)),
            pl.BlockSpec((MROWS, 128), lambda i: (i, 0)),
            full(w0u), full(w0i), full(b0r), full(w1b), full(b1r),
            full(w2b), full(b2r), full(whb), full(wg2),
        ],
        out_specs=[
            pl.BlockSpec((GROWS, 2), lambda i: (i, 0)),
            pl.BlockSpec((MROWS, 4), lambda i: (i, 0)),
        ],
        out_shape=[
            jax.ShapeDtypeStruct((BATCH // 2, 2), jnp.float32),
            jax.ShapeDtypeStruct((BATCH // 4, 4), jnp.float32),
        ],
    )(gu2, gi2, mu2, mi2, w0u, w0i, b0r, w1b, b1r, w2b, b2r, whb, wg2)
    return og, om


def kernel(user_ids, item_ids, gmf_user_emb, gmf_item_emb, mlp_user_emb,
           mlp_item_emb, W0, b0, W1, b1, W2, b2, Wout, bout):
    gu2, gi2, mu2, mi2 = _sc_gather(user_ids, item_ids, gmf_user_emb,
                                    gmf_item_emb, mlp_user_emb, mlp_item_emb)
    og, om = _tc_dense(gu2, gi2, mu2, mi2,
                       W0, b0, W1, b1, W2, b2, Wout, bout)
    return jax.nn.sigmoid(og.reshape(BATCH) + om.reshape(BATCH) + bout[0])
